# speedup vs baseline: 7.2277x; 7.2277x over previous
import functools

import jax
import jax.numpy as jnp
from jax import lax
from jax.experimental import pallas as pl
from jax.experimental.pallas import tpu as pltpu
from jax.experimental.pallas import tpu_sc as plsc

N = 10000
D = 128
NPAD = 10112
NC = 2
NS = 16
NW = NC * NS
L = 16
CHUNK = 128
CPW = 80
EPW = CPW * CHUNK
EP = EPW * NW
RPT = NPAD // NS
DUMP = N

_mesh = plsc.VectorSubcoreMesh(core_axis_name="c", subcore_axis_name="s")


@functools.partial(
    pl.kernel,
    out_type=jax.ShapeDtypeStruct((NC * NPAD, L), jnp.float32),
    mesh=_mesh,
    scratch_types=[
        pltpu.VMEM((CPW, CHUNK), jnp.int32),
        pltpu.VMEM((CHUNK, L), jnp.float32),
        pltpu.VMEM_SHARED((NPAD, L), jnp.float32),
    ],
)
def _deg_kernel(dstw, zeros16, deg_out, dst_v, ones_v, acc):
    c = lax.axis_index("c")
    s = lax.axis_index("s")
    wid = s * NC + c
    r0 = s * RPT
    pltpu.sync_copy(dstw.at[wid], dst_v)
    for i in range(CHUNK):
        ones_v[i] = jnp.ones((L,), jnp.float32)
    pltpu.sync_copy(zeros16.at[pl.ds(r0, RPT)], acc.at[pl.ds(r0, RPT)])
    plsc.subcore_barrier()

    def body(j, carry):
        pltpu.sync_copy(ones_v, acc.at[dst_v.at[j]], add=True)
        return carry

    lax.fori_loop(0, CPW, body, 0)
    plsc.subcore_barrier()
    pltpu.sync_copy(acc.at[pl.ds(r0, RPT)], deg_out.at[pl.ds(c * NPAD + r0, RPT)])


@functools.partial(
    pl.kernel,
    out_type=jax.ShapeDtypeStruct((NC * NPAD, D), jnp.float32),
    mesh=_mesh,
    scratch_types=[
        pltpu.VMEM((CPW, CHUNK), jnp.int32),
        pltpu.VMEM((CPW, CHUNK), jnp.int32),
        pltpu.VMEM((CHUNK, D), jnp.float32),
        pltpu.VMEM_SHARED((NPAD, D), jnp.float32),
        pltpu.SemaphoreType.DMA,
    ],
)
def _agg_kernel(h, srcw, dstw, zrows, acc_out, src_v, dst_v, buf, acc, sem):
    c = lax.axis_index("c")
    s = lax.axis_index("s")
    wid = s * NC + c
    r0 = s * RPT
    pltpu.sync_copy(srcw.at[wid], src_v)
    pltpu.sync_copy(dstw.at[wid], dst_v)

    @pl.when(c == 0)
    def _():
        pltpu.sync_copy(h.at[pl.ds(r0, RPT)], acc.at[pl.ds(r0, RPT)])

    @pl.when(c == 1)
    def _():
        pltpu.sync_copy(zrows.at[pl.ds(r0, RPT)], acc.at[pl.ds(r0, RPT)])

    plsc.subcore_barrier()

    def body(j, carry):
        pltpu.async_copy(h.at[src_v.at[j]], buf, sem).wait()
        pltpu.sync_copy(buf, acc.at[dst_v.at[j]], add=True)
        return carry

    lax.fori_loop(0, CPW, body, 0)
    plsc.subcore_barrier()
    pltpu.sync_copy(acc.at[pl.ds(r0, RPT)], acc_out.at[pl.ds(c * NPAD + r0, RPT)])


BLK = 1264
GRID = NPAD // BLK


def _dinv(d0_ref, d1_ref):
    deg = d0_ref[0, :, 0:1] + d1_ref[0, :, 0:1] + 1.0
    return lax.rsqrt(deg)


def _tc1_body(d0_ref, d1_ref, x_ref, w_ref, h_ref):
    dinv = _dinv(d0_ref, d1_ref)
    h_ref[...] = jnp.dot(
        x_ref[...], w_ref[...],
        preferred_element_type=jnp.float32, precision=lax.Precision.HIGHEST,
    ) * dinv


def _tc2_body(a0_ref, a1_ref, d0_ref, d1_ref, b_ref, w_ref, h_ref):
    dinv = _dinv(d0_ref, d1_ref)
    g = jnp.maximum((a0_ref[0] + a1_ref[0]) * dinv + b_ref[...], 0.0)
    h_ref[...] = jnp.dot(
        g, w_ref[...],
        preferred_element_type=jnp.float32, precision=lax.Precision.HIGHEST,
    ) * dinv


def _tc3_body(a0_ref, a1_ref, d0_ref, d1_ref, b_ref, out_ref):
    dinv = _dinv(d0_ref, d1_ref)
    out_ref[...] = (a0_ref[0] + a1_ref[0]) * dinv + b_ref[...]


_deg_spec0 = pl.BlockSpec((1, BLK, L), lambda i: (0, i, 0))
_deg_spec1 = pl.BlockSpec((1, BLK, L), lambda i: (1, i, 0))
_acc_spec0 = pl.BlockSpec((1, BLK, D), lambda i: (0, i, 0))
_acc_spec1 = pl.BlockSpec((1, BLK, D), lambda i: (1, i, 0))
_row_spec = pl.BlockSpec((BLK, D), lambda i: (i, 0))
_w_spec = pl.BlockSpec((D, D), lambda i: (0, 0))
_b_spec = pl.BlockSpec((1, D), lambda i: (0, 0))
_rows_out = jax.ShapeDtypeStruct((NPAD, D), jnp.float32)


@jax.jit
def kernel(x, edge_index, W1, b1, W2, b2):
    n_edges = edge_index.shape[1]
    npad_e = EP - n_edges
    src = jnp.concatenate(
        [edge_index[0].astype(jnp.int32), jnp.zeros((npad_e,), jnp.int32)]
    ).reshape(NW, CPW, CHUNK)
    dst = jnp.concatenate(
        [edge_index[1].astype(jnp.int32), jnp.full((npad_e,), DUMP, jnp.int32)]
    ).reshape(NW, CPW, CHUNK)
    x_pad = jnp.concatenate([x, jnp.zeros((NPAD - N, D), x.dtype)])
    zeros16 = jnp.zeros((NPAD, L), jnp.float32)
    zrows = jnp.zeros((NPAD, D), jnp.float32)
    b1r = b1.reshape(1, D)
    b2r = b2.reshape(1, D)

    degs = _deg_kernel(dst, zeros16).reshape(NC, NPAD, L)

    h1 = pl.pallas_call(
        _tc1_body,
        grid=(GRID,),
        in_specs=[_deg_spec0, _deg_spec1, _row_spec, _w_spec],
        out_specs=_row_spec,
        out_shape=_rows_out,
    )(degs, degs, x_pad, W1)

    agg1 = _agg_kernel(h1, src, dst, zrows).reshape(NC, NPAD, D)

    h2 = pl.pallas_call(
        _tc2_body,
        grid=(GRID,),
        in_specs=[_acc_spec0, _acc_spec1, _deg_spec0, _deg_spec1, _b_spec, _w_spec],
        out_specs=_row_spec,
        out_shape=_rows_out,
    )(agg1, agg1, degs, degs, b1r, W2)

    agg2 = _agg_kernel(h2, src, dst, zrows).reshape(NC, NPAD, D)

    out = pl.pallas_call(
        _tc3_body,
        grid=(GRID,),
        in_specs=[_acc_spec0, _acc_spec1, _deg_spec0, _deg_spec1, _b_spec],
        out_specs=_row_spec,
        out_shape=_rows_out,
    )(agg2, agg2, degs, degs, b2r)

    return out[:N]
